# Initial kernel scaffold; baseline (speedup 1.0000x reference)
import jax
import jax.numpy as jnp
from jax.experimental import pallas as pl


def kernel(embed, triplets, labels, w_relation):
    raise NotImplementedError("write your pallas kernel here")


# speedup vs baseline: 4.4916x; 4.4916x over previous
import functools

import jax
import jax.numpy as jnp
from jax import lax
from jax.experimental import pallas as pl
from jax.experimental.pallas import tpu as pltpu
from jax.experimental.pallas import tpu_sc as plsc

N_NODES = 10000
N_TRIPLETS = 320000
H_DIM = 128
REG = 0.01

NC = 2
NS = 16
NW = NC * NS
PER_W = N_TRIPLETS // NW
C = 80
NCHUNK = PER_W // C
G = C // 16


def _sc_scores(embed, w_relation, s_idx, r_idx, o_idx):
    mesh = plsc.VectorSubcoreMesh(
        core_axis_name="c", subcore_axis_name="s", num_cores=NC, num_subcores=NS
    )

    @functools.partial(
        pl.kernel,
        out_type=jax.ShapeDtypeStruct((N_TRIPLETS,), jnp.float32),
        mesh=mesh,
        compiler_params=pltpu.CompilerParams(needs_layout_passes=False),
        scratch_types=[
            pltpu.VMEM((PER_W,), jnp.int32),
            pltpu.VMEM((PER_W,), jnp.int32),
            pltpu.VMEM((PER_W,), jnp.int32),
            pltpu.VMEM((C, H_DIM), jnp.float32),
            pltpu.VMEM((C, H_DIM), jnp.float32),
            pltpu.VMEM((C, H_DIM), jnp.float32),
            pltpu.VMEM((PER_W,), jnp.float32),
            pltpu.VMEM((256,), jnp.float32),
            pltpu.SemaphoreType.DMA,
        ],
    )
    def scores_kernel(embed_hbm, w_hbm, sidx_hbm, ridx_hbm, oidx_hbm, out_hbm,
                      sidx_v, ridx_v, oidx_v, srow, rrow, orow, score_v, tmp_v, sem):
        wid = lax.axis_index("c") * NS + lax.axis_index("s")
        base = wid * PER_W
        pltpu.sync_copy(sidx_hbm.at[pl.ds(base, PER_W)], sidx_v)
        pltpu.sync_copy(ridx_hbm.at[pl.ds(base, PER_W)], ridx_v)
        pltpu.sync_copy(oidx_hbm.at[pl.ds(base, PER_W)], oidx_v)

        def chunk_body(k, carry):
            koff = k * C
            cs = pltpu.async_copy(embed_hbm.at[sidx_v.at[pl.ds(koff, C)]], srow, sem)
            cr = pltpu.async_copy(w_hbm.at[ridx_v.at[pl.ds(koff, C)]], rrow, sem)
            co = pltpu.async_copy(embed_hbm.at[oidx_v.at[pl.ds(koff, C)]], orow, sem)
            cs.wait()
            cr.wait()
            co.wait()
            lanes = lax.iota(jnp.int32, 16)
            for g in range(G):
                def rbody(j, carry2):
                    row = g * 16 + j
                    p = jnp.zeros((16,), jnp.float32)
                    for t in range(H_DIM // 16):
                        sl = pl.ds(t * 16, 16)
                        p = p + srow[row, sl] * rrow[row, sl] * orow[row, sl]
                    plsc.store_scatter(tmp_v, [lanes * 16 + j], p)
                    return carry2

                lax.fori_loop(0, 16, rbody, 0, unroll=4)
                sg = tmp_v[pl.ds(0, 16)]
                for l in range(1, 16):
                    sg = sg + tmp_v[pl.ds(l * 16, 16)]
                score_v[pl.ds(koff + g * 16, 16)] = sg
            return carry

        lax.fori_loop(0, NCHUNK, chunk_body, 0)
        pltpu.sync_copy(score_v, out_hbm.at[pl.ds(base, PER_W)])

    return scores_kernel(embed, w_relation, s_idx, r_idx, o_idx)


def _loss_body(s_ref, l_ref, e_ref, w_ref, o_ref):
    s = s_ref[...]
    lbl = l_ref[...]
    t = jnp.maximum(s, 0.0) - s * lbl + jnp.log1p(jnp.exp(-jnp.abs(s)))
    pred = jnp.sum(t) * (1.0 / N_TRIPLETS)
    reg = (jnp.sum(e_ref[...] ** 2) + jnp.sum(w_ref[...] ** 2)) * (
        1.0 / (N_NODES * H_DIM)
    )
    o_ref[...] = (pred + REG * reg).reshape(1, 1)


def _tc_loss(scores2d, labels2d, embed, w_relation):
    return pl.pallas_call(
        _loss_body,
        out_shape=jax.ShapeDtypeStruct((1, 1), jnp.float32),
    )(scores2d, labels2d, embed, w_relation)


def kernel(embed, triplets, labels, w_relation):
    s_idx = triplets[:, 0]
    r_idx = triplets[:, 1]
    o_idx = triplets[:, 2]
    scores = _sc_scores(embed, w_relation, s_idx, r_idx, o_idx)
    rows = N_TRIPLETS // H_DIM
    loss = _tc_loss(
        scores.reshape(rows, H_DIM), labels.reshape(rows, H_DIM), embed, w_relation
    )
    return loss[0, 0]


# speedup vs baseline: 7.1817x; 1.5989x over previous
import functools

import jax
import jax.numpy as jnp
from jax import lax
from jax.experimental import pallas as pl
from jax.experimental.pallas import tpu as pltpu
from jax.experimental.pallas import tpu_sc as plsc

N_NODES = 10000
N_TRIPLETS = 320000
H_DIM = 128
REG = 0.01

NC = 2
NS = 16
NW = NC * NS
PER_W = N_TRIPLETS // NW
C = 80
NCHUNK = PER_W // C
G = C // 16


def _sc_scores(embed, w_relation, s_idx, r_idx, o_idx):
    mesh = plsc.VectorSubcoreMesh(
        core_axis_name="c", subcore_axis_name="s", num_cores=NC, num_subcores=NS
    )

    @functools.partial(
        pl.kernel,
        out_type=jax.ShapeDtypeStruct((N_TRIPLETS,), jnp.float32),
        mesh=mesh,
        compiler_params=pltpu.CompilerParams(needs_layout_passes=False),
        scratch_types=[
            pltpu.VMEM((PER_W,), jnp.int32),
            pltpu.VMEM((PER_W,), jnp.int32),
            pltpu.VMEM((PER_W,), jnp.int32),
            pltpu.VMEM((C, H_DIM), jnp.float32),
            pltpu.VMEM((C, H_DIM), jnp.float32),
            pltpu.VMEM((C, H_DIM), jnp.float32),
            pltpu.VMEM((C, H_DIM), jnp.float32),
            pltpu.VMEM((C, H_DIM), jnp.float32),
            pltpu.VMEM((C, H_DIM), jnp.float32),
            pltpu.VMEM((PER_W,), jnp.float32),
            pltpu.VMEM((256,), jnp.float32),
            pltpu.SemaphoreType.DMA,
            pltpu.SemaphoreType.DMA,
        ],
    )
    def scores_kernel(embed_hbm, w_hbm, sidx_hbm, ridx_hbm, oidx_hbm, out_hbm,
                      sidx_v, ridx_v, oidx_v,
                      srow_a, rrow_a, orow_a, srow_b, rrow_b, orow_b,
                      score_v, tmp_v, sem_a, sem_b):
        wid = lax.axis_index("c") * NS + lax.axis_index("s")
        base = wid * PER_W
        pltpu.sync_copy(sidx_hbm.at[pl.ds(base, PER_W)], sidx_v)
        pltpu.sync_copy(ridx_hbm.at[pl.ds(base, PER_W)], ridx_v)
        pltpu.sync_copy(oidx_hbm.at[pl.ds(base, PER_W)], oidx_v)
        bufs = ((srow_a, rrow_a, orow_a, sem_a), (srow_b, rrow_b, orow_b, sem_b))

        def issue(k, b):
            koff = k * C
            sr, rr, outr, sem = b
            pltpu.async_copy(embed_hbm.at[sidx_v.at[pl.ds(koff, C)]], sr, sem)
            pltpu.async_copy(w_hbm.at[ridx_v.at[pl.ds(koff, C)]], rr, sem)
            pltpu.async_copy(embed_hbm.at[oidx_v.at[pl.ds(koff, C)]], outr, sem)

        def drain(b):
            sr, rr, outr, sem = b
            pltpu.make_async_copy(embed_hbm.at[sidx_v.at[pl.ds(0, C)]], sr, sem).wait()
            pltpu.make_async_copy(w_hbm.at[ridx_v.at[pl.ds(0, C)]], rr, sem).wait()
            pltpu.make_async_copy(embed_hbm.at[oidx_v.at[pl.ds(0, C)]], outr, sem).wait()

        def compute(k, b):
            sr, rr, outr, _ = b
            koff = k * C
            lanes = lax.iota(jnp.int32, 16)
            for g in range(G):
                def rbody(j, carry2):
                    row = g * 16 + j
                    p = jnp.zeros((16,), jnp.float32)
                    for t in range(H_DIM // 16):
                        sl = pl.ds(t * 16, 16)
                        p = p + sr[row, sl] * rr[row, sl] * outr[row, sl]
                    plsc.store_scatter(tmp_v, [lanes * 16 + j], p)
                    return carry2

                lax.fori_loop(0, 16, rbody, 0, unroll=4)
                sg = tmp_v[pl.ds(0, 16)]
                for l in range(1, 16):
                    sg = sg + tmp_v[pl.ds(l * 16, 16)]
                score_v[pl.ds(koff + g * 16, 16)] = sg

        issue(0, bufs[0])

        def body(i, carry):
            k0 = 2 * i
            drain(bufs[0])
            issue(k0 + 1, bufs[1])
            compute(k0, bufs[0])
            drain(bufs[1])
            issue(k0 + 2, bufs[0])
            compute(k0 + 1, bufs[1])
            return carry

        lax.fori_loop(0, (NCHUNK - 1) // 2, body, 0)
        drain(bufs[0])
        compute(NCHUNK - 1, bufs[0])
        pltpu.sync_copy(score_v, out_hbm.at[pl.ds(base, PER_W)])

    return scores_kernel(embed, w_relation, s_idx, r_idx, o_idx)


def _loss_body(s_ref, l_ref, e_ref, w_ref, o_ref):
    s = s_ref[...]
    lbl = l_ref[...]
    t = jnp.maximum(s, 0.0) - s * lbl + jnp.log1p(jnp.exp(-jnp.abs(s)))
    pred = jnp.sum(t) * (1.0 / N_TRIPLETS)
    reg = (jnp.sum(e_ref[...] ** 2) + jnp.sum(w_ref[...] ** 2)) * (
        1.0 / (N_NODES * H_DIM)
    )
    o_ref[...] = (pred + REG * reg).reshape(1, 1)


def _tc_loss(scores2d, labels2d, embed, w_relation):
    return pl.pallas_call(
        _loss_body,
        out_shape=jax.ShapeDtypeStruct((1, 1), jnp.float32),
    )(scores2d, labels2d, embed, w_relation)


def kernel(embed, triplets, labels, w_relation):
    s_idx = triplets[:, 0]
    r_idx = triplets[:, 1]
    o_idx = triplets[:, 2]
    scores = _sc_scores(embed, w_relation, s_idx, r_idx, o_idx)
    rows = N_TRIPLETS // H_DIM
    loss = _tc_loss(
        scores.reshape(rows, H_DIM), labels.reshape(rows, H_DIM), embed, w_relation
    )
    return loss[0, 0]


# speedup vs baseline: 7.4918x; 1.0432x over previous
import functools

import jax
import jax.numpy as jnp
from jax import lax
from jax.experimental import pallas as pl
from jax.experimental.pallas import tpu as pltpu
from jax.experimental.pallas import tpu_sc as plsc

N_NODES = 10000
N_TRIPLETS = 320000
H_DIM = 128
REG = 0.01

NC = 2
NS = 16
NW = NC * NS
PER_W = N_TRIPLETS // NW
C = 80
NCHUNK = PER_W // C
G = C // 16


def _sc_scores(embed, w_relation, s_idx, r_idx, o_idx):
    mesh = plsc.VectorSubcoreMesh(
        core_axis_name="c", subcore_axis_name="s", num_cores=NC, num_subcores=NS
    )

    @functools.partial(
        pl.kernel,
        out_type=jax.ShapeDtypeStruct((N_TRIPLETS,), jnp.float32),
        mesh=mesh,
        compiler_params=pltpu.CompilerParams(needs_layout_passes=False),
        scratch_types=[
            pltpu.VMEM((PER_W,), jnp.int32),
            pltpu.VMEM((PER_W,), jnp.int32),
            pltpu.VMEM((PER_W,), jnp.int32),
            pltpu.VMEM((C, H_DIM), jnp.float32),
            pltpu.VMEM((C, H_DIM), jnp.float32),
            pltpu.VMEM((C, H_DIM), jnp.float32),
            pltpu.VMEM((C, H_DIM), jnp.float32),
            pltpu.VMEM((C, H_DIM), jnp.float32),
            pltpu.VMEM((C, H_DIM), jnp.float32),
            pltpu.VMEM((PER_W,), jnp.float32),
            pltpu.VMEM((16 * C,), jnp.float32),
            pltpu.SemaphoreType.DMA,
            pltpu.SemaphoreType.DMA,
        ],
    )
    def scores_kernel(embed_hbm, w_hbm, sidx_hbm, ridx_hbm, oidx_hbm, out_hbm,
                      sidx_v, ridx_v, oidx_v,
                      srow_a, rrow_a, orow_a, srow_b, rrow_b, orow_b,
                      score_v, tmp_v, sem_a, sem_b):
        wid = lax.axis_index("c") * NS + lax.axis_index("s")
        base = wid * PER_W
        pltpu.sync_copy(sidx_hbm.at[pl.ds(base, PER_W)], sidx_v)
        pltpu.sync_copy(ridx_hbm.at[pl.ds(base, PER_W)], ridx_v)
        pltpu.sync_copy(oidx_hbm.at[pl.ds(base, PER_W)], oidx_v)
        bufs = ((srow_a, rrow_a, orow_a, sem_a), (srow_b, rrow_b, orow_b, sem_b))

        def issue(k, b):
            koff = k * C
            sr, rr, outr, sem = b
            pltpu.async_copy(embed_hbm.at[sidx_v.at[pl.ds(koff, C)]], sr, sem)
            pltpu.async_copy(w_hbm.at[ridx_v.at[pl.ds(koff, C)]], rr, sem)
            pltpu.async_copy(embed_hbm.at[oidx_v.at[pl.ds(koff, C)]], outr, sem)

        def drain(b):
            sr, rr, outr, sem = b
            pltpu.make_async_copy(embed_hbm.at[sidx_v.at[pl.ds(0, C)]], sr, sem).wait()
            pltpu.make_async_copy(w_hbm.at[ridx_v.at[pl.ds(0, C)]], rr, sem).wait()
            pltpu.make_async_copy(embed_hbm.at[oidx_v.at[pl.ds(0, C)]], outr, sem).wait()

        def compute(k, b):
            sr, rr, outr, _ = b
            koff = k * C
            lanes = lax.iota(jnp.int32, 16)

            @functools.partial(plsc.parallel_loop, 0, C, unroll=4)
            def _(j):
                p = jnp.zeros((16,), jnp.float32)
                for t in range(H_DIM // 16):
                    sl = pl.ds(t * 16, 16)
                    p = p + sr[j, sl] * rr[j, sl] * outr[j, sl]
                plsc.store_scatter(tmp_v, [lanes * C + j], p)

            for g in range(G):
                sg = tmp_v[pl.ds(g * 16, 16)]
                for l in range(1, 16):
                    sg = sg + tmp_v[pl.ds(l * C + g * 16, 16)]
                score_v[pl.ds(koff + g * 16, 16)] = sg

        issue(0, bufs[0])

        def body(i, carry):
            k0 = 2 * i
            drain(bufs[0])
            issue(k0 + 1, bufs[1])
            compute(k0, bufs[0])
            drain(bufs[1])
            issue(k0 + 2, bufs[0])
            compute(k0 + 1, bufs[1])
            return carry

        lax.fori_loop(0, (NCHUNK - 1) // 2, body, 0)
        drain(bufs[0])
        compute(NCHUNK - 1, bufs[0])
        pltpu.sync_copy(score_v, out_hbm.at[pl.ds(base, PER_W)])

    return scores_kernel(embed, w_relation, s_idx, r_idx, o_idx)


def _loss_body(s_ref, l_ref, e_ref, w_ref, o_ref):
    s = s_ref[...]
    lbl = l_ref[...]
    t = jnp.maximum(s, 0.0) - s * lbl + jnp.log1p(jnp.exp(-jnp.abs(s)))
    pred = jnp.sum(t) * (1.0 / N_TRIPLETS)
    reg = (jnp.sum(e_ref[...] ** 2) + jnp.sum(w_ref[...] ** 2)) * (
        1.0 / (N_NODES * H_DIM)
    )
    o_ref[...] = (pred + REG * reg).reshape(1, 1)


def _tc_loss(scores2d, labels2d, embed, w_relation):
    return pl.pallas_call(
        _loss_body,
        out_shape=jax.ShapeDtypeStruct((1, 1), jnp.float32),
    )(scores2d, labels2d, embed, w_relation)


def kernel(embed, triplets, labels, w_relation):
    s_idx = triplets[:, 0]
    r_idx = triplets[:, 1]
    o_idx = triplets[:, 2]
    scores = _sc_scores(embed, w_relation, s_idx, r_idx, o_idx)
    rows = N_TRIPLETS // H_DIM
    loss = _tc_loss(
        scores.reshape(rows, H_DIM), labels.reshape(rows, H_DIM), embed, w_relation
    )
    return loss[0, 0]
